# Initial kernel scaffold; baseline (speedup 1.0000x reference)
import jax
import jax.numpy as jnp
from jax.experimental import pallas as pl


def kernel(x, edge_index, W1, b1, W2, b2, W3, b3):
    raise NotImplementedError("write your pallas kernel here")


# speedup vs baseline: 4.2976x; 4.2976x over previous
import functools

import jax
import jax.numpy as jnp
from jax import lax
from jax.experimental import pallas as pl
from jax.experimental.pallas import tpu as pltpu
from jax.experimental.pallas import tpu_sc as plsc

N, E, F, H1, H2, C = 10000, 320000, 128, 100, 100, 16
DP = 128
NPAD = 10240
NC, NS = 2, 16
CHUNK = 128
CHUNKS_PER_TILE = 79
EROWS = NC * NS * CHUNKS_PER_TILE
EPAD = EROWS * CHUNK
ROWS_PER_TILE = NPAD // NS
MBLK = 1000


def _agg_body(support, src2d, dst2d, zeros, out, srcb, dstb, rows, agg, sem):
    c = lax.axis_index("c")
    s = lax.axis_index("s")
    pltpu.sync_copy(zeros, agg.at[pl.ds(s * ROWS_PER_TILE, ROWS_PER_TILE)])
    plsc.subcore_barrier()
    base = (c * NS + s) * CHUNKS_PER_TILE

    def step(j, carry):
        pltpu.sync_copy(src2d.at[base + j], srcb)
        pltpu.sync_copy(dst2d.at[base + j], dstb)
        pltpu.async_copy(support.at[srcb], rows, sem).wait()
        pltpu.sync_copy(rows, agg.at[dstb], add=True)
        return carry

    lax.fori_loop(0, CHUNKS_PER_TILE, step, 0)
    plsc.subcore_barrier()
    sl = pl.ds(s * ROWS_PER_TILE, ROWS_PER_TILE)
    pltpu.sync_copy(agg.at[sl], out.at[c, sl])


_aggregate = functools.partial(
    pl.kernel,
    out_type=jax.ShapeDtypeStruct((NC, NPAD, DP), jnp.float32),
    mesh=plsc.VectorSubcoreMesh(core_axis_name="c", subcore_axis_name="s"),
    scratch_types=[
        pltpu.VMEM((CHUNK,), jnp.int32),
        pltpu.VMEM((CHUNK,), jnp.int32),
        pltpu.VMEM((CHUNK, DP), jnp.float32),
        pltpu.VMEM_SHARED((NPAD, DP), jnp.float32),
        pltpu.SemaphoreType.DMA,
    ],
)(_agg_body)


def _mm_body(x_ref, w_ref, o_ref):
    o_ref[...] = jnp.dot(x_ref[...], w_ref[...],
                         preferred_element_type=jnp.float32)


def _combine_mm_body(p_ref, b_ref, w_ref, o_ref):
    h = jnp.maximum(p_ref[0] + p_ref[1] + b_ref[...], 0.0)
    o_ref[...] = jnp.dot(h, w_ref[...], preferred_element_type=jnp.float32)


def _head_body(p_ref, b2_ref, w3_ref, b3_ref, o_ref):
    h = jnp.maximum(p_ref[0] + p_ref[1] + b2_ref[...], 0.0)
    logits = jnp.dot(h, w3_ref[...], preferred_element_type=jnp.float32)
    logits = logits + b3_ref[...]
    m = jnp.max(logits, axis=1, keepdims=True)
    lse = jnp.log(jnp.sum(jnp.exp(logits - m), axis=1, keepdims=True)) + m
    o_ref[...] = logits - lse


def _mm(x, w):
    grid = N // MBLK
    return pl.pallas_call(
        _mm_body,
        grid=(grid,),
        in_specs=[
            pl.BlockSpec((MBLK, F), lambda i: (i, 0)),
            pl.BlockSpec((F, DP), lambda i: (0, 0)),
        ],
        out_specs=pl.BlockSpec((MBLK, DP), lambda i: (i, 0)),
        out_shape=jax.ShapeDtypeStruct((N, DP), jnp.float32),
    )(x, w)


def _combine_mm(parts, b, w):
    grid = N // MBLK
    return pl.pallas_call(
        _combine_mm_body,
        grid=(grid,),
        in_specs=[
            pl.BlockSpec((NC, MBLK, DP), lambda i: (0, i, 0)),
            pl.BlockSpec((1, DP), lambda i: (0, 0)),
            pl.BlockSpec((DP, DP), lambda i: (0, 0)),
        ],
        out_specs=pl.BlockSpec((MBLK, DP), lambda i: (i, 0)),
        out_shape=jax.ShapeDtypeStruct((N, DP), jnp.float32),
    )(parts, b, w)


def _head(parts, b2, w3, b3):
    grid = N // MBLK
    return pl.pallas_call(
        _head_body,
        grid=(grid,),
        in_specs=[
            pl.BlockSpec((NC, MBLK, DP), lambda i: (0, i, 0)),
            pl.BlockSpec((1, DP), lambda i: (0, 0)),
            pl.BlockSpec((DP, C), lambda i: (0, 0)),
            pl.BlockSpec((1, C), lambda i: (0, 0)),
        ],
        out_specs=pl.BlockSpec((MBLK, C), lambda i: (i, 0)),
        out_shape=jax.ShapeDtypeStruct((N, C), jnp.float32),
    )(parts, b2, w3, b3)


def kernel(x, edge_index, W1, b1, W2, b2, W3, b3):
    src = edge_index[0]
    dst = edge_index[1]

    w1p = jnp.zeros((F, DP), jnp.float32).at[:, :H1].set(W1)
    b1p = jnp.zeros((1, DP), jnp.float32).at[0, :H1].set(b1)
    w2p = jnp.zeros((DP, DP), jnp.float32).at[:H1, :H2].set(W2)
    b2p = jnp.zeros((1, DP), jnp.float32).at[0, :H2].set(b2)
    w3p = jnp.zeros((DP, C), jnp.float32).at[:H2, :].set(W3)
    b3p = b3.reshape(1, C)

    srcp = jnp.concatenate(
        [src, jnp.zeros((EPAD - E,), jnp.int32)]).reshape(EROWS, CHUNK)
    dstp = jnp.concatenate(
        [dst, jnp.full((EPAD - E,), N, jnp.int32)]).reshape(EROWS, CHUNK)
    zeros = jnp.zeros((ROWS_PER_TILE, DP), jnp.float32)

    support1 = _mm(x, w1p)
    parts1 = _aggregate(support1, srcp, dstp, zeros)
    support2 = _combine_mm(parts1, b1p, w2p)
    parts2 = _aggregate(support2, srcp, dstp, zeros)
    return _head(parts2, b2p, w3p, b3p)
